# Initial kernel scaffold; baseline (speedup 1.0000x reference)
import jax
import jax.numpy as jnp
from jax.experimental import pallas as pl


def kernel(x, edge_index, batch, W1, b1, gamma1, beta1, W2, b2, gamma2, beta2, W3, b3, fc1_W, fc1_b, fc2_W, fc2_b):
    raise NotImplementedError("write your pallas kernel here")


# speedup vs baseline: 10.9637x; 10.9637x over previous
import functools
import jax
import jax.numpy as jnp
from jax import lax
from jax.experimental import pallas as pl
from jax.experimental.pallas import tpu as pltpu
from jax.experimental.pallas import tpu_sc as plsc

N = 10000
E = 320000
F = 128
H = 128
DE = 64
DO = 1
G = 128

NC = 2
NS = 16
NW = NC * NS
EW = E // NW
CH = 80
NCHUNK = EW // CH
RPT = 624
NTAIL = N - NS * RPT
NDEG = 10240
DPT = NDEG // NS

_SELU_ALPHA = 1.6732632423543772
_SELU_SCALE = 1.0507009873554805


def _selu(x):
    return _SELU_SCALE * jnp.where(
        x > 0, x, _SELU_ALPHA * (jnp.exp(jnp.minimum(x, 0.0)) - 1.0))


@functools.cache
def _mesh():
    return plsc.VectorSubcoreMesh(core_axis_name="c", subcore_axis_name="s",
                                  num_cores=NC, num_subcores=NS)


def _deg_body(dst_hbm, out_hbm, idx_v, ones_v, z_v, acc):
    c = lax.axis_index("c")
    s = lax.axis_index("s")
    wid = s * NC + c

    @pl.loop(0, DPT, step=16)
    def _(i):
        z_v[pl.ds(i, 16)] = jnp.zeros((16,), jnp.float32)

    pltpu.sync_copy(z_v, acc.at[pl.ds(s * DPT, DPT)])

    @pl.loop(0, CH, step=16)
    def _(i):
        ones_v[pl.ds(i, 16)] = jnp.ones((16,), jnp.float32)

    plsc.subcore_barrier()

    @pl.loop(0, NCHUNK)
    def _(i):
        off = pl.multiple_of(wid * EW + i * CH, 8)
        pltpu.sync_copy(dst_hbm.at[pl.ds(off, CH)], idx_v)
        pltpu.sync_copy(ones_v, acc.at[idx_v], add=True)

    plsc.subcore_barrier()
    pltpu.sync_copy(
        acc.at[pl.ds(s * DPT, DPT)], out_hbm.at[c, pl.ds(s * DPT, DPT)]
    )


@functools.cache
def _deg_kernel():
    return pl.kernel(
        _deg_body,
        out_type=jax.ShapeDtypeStruct((NC, NDEG), jnp.float32),
        mesh=_mesh(),
        scratch_types=[
            pltpu.VMEM((CH,), jnp.int32),
            pltpu.VMEM((CH,), jnp.float32),
            pltpu.VMEM((DPT,), jnp.float32),
            pltpu.VMEM_SHARED((NDEG,), jnp.float32),
        ],
    )


def _agg_body(hs_hbm, src_hbm, dst_hbm, out_hbm, idx_s, idx_d, rows, acc, sem):
    c = lax.axis_index("c")
    s = lax.axis_index("s")
    wid = s * NC + c

    pltpu.sync_copy(
        hs_hbm.at[pl.ds(s * RPT, RPT)], acc.at[pl.ds(s * RPT, RPT)]
    )

    @pl.when(s == NS - 1)
    def _():
        pltpu.sync_copy(
            hs_hbm.at[pl.ds(NS * RPT, NTAIL)], acc.at[pl.ds(NS * RPT, NTAIL)]
        )

    plsc.subcore_barrier()

    @pl.loop(0, NCHUNK)
    def _(i):
        off = pl.multiple_of(wid * EW + i * CH, 8)
        pltpu.sync_copy(src_hbm.at[pl.ds(off, CH)], idx_s)
        pltpu.sync_copy(dst_hbm.at[pl.ds(off, CH)], idx_d)
        pltpu.async_copy(hs_hbm.at[idx_s], rows, sem).wait()
        pltpu.sync_copy(rows, acc.at[idx_d], add=True)

    plsc.subcore_barrier()
    pltpu.sync_copy(
        acc.at[pl.ds(s * RPT, RPT)], out_hbm.at[c, pl.ds(s * RPT, RPT)]
    )

    @pl.when(s == NS - 1)
    def _():
        pltpu.sync_copy(
            acc.at[pl.ds(NS * RPT, NTAIL)],
            out_hbm.at[c, pl.ds(NS * RPT, NTAIL)],
        )


@functools.cache
def _agg_kernel():
    return pl.kernel(
        _agg_body,
        out_type=jax.ShapeDtypeStruct((NC, N, H), jnp.float32),
        mesh=_mesh(),
        scratch_types=[
            pltpu.VMEM((CH,), jnp.int32),
            pltpu.VMEM((CH,), jnp.int32),
            pltpu.VMEM((CH, H), jnp.float32),
            pltpu.VMEM_SHARED((N, H), jnp.float32),
            pltpu.SemaphoreType.DMA,
        ],
    )


RB = 1000
_PREC = lax.Precision.HIGHEST


def _dinv_body(dg_ref, out_ref):
    out_ref[...] = lax.rsqrt(dg_ref[0:1, :] + dg_ref[1:2, :] + 1.0)


def _dinv(deg2):
    return pl.pallas_call(
        _dinv_body,
        out_shape=jax.ShapeDtypeStruct((1, NDEG), jnp.float32),
    )(deg2)


def _mm_body(h_ref, W_ref, a_ref, c_ref, dinv_ref, out_ref):
    hb = h_ref[...] * a_ref[...] + c_ref[...]
    p = jnp.dot(hb, W_ref[...], preferred_element_type=jnp.float32,
                precision=_PREC)
    out_ref[...] = p * dinv_ref[...]


def _mm(h, W, a, c, dinv):
    return pl.pallas_call(
        _mm_body,
        grid=(N // RB,),
        in_specs=[
            pl.BlockSpec((RB, H), lambda i: (i, 0)),
            pl.BlockSpec((H, H), lambda i: (0, 0)),
            pl.BlockSpec((1, H), lambda i: (0, 0)),
            pl.BlockSpec((1, H), lambda i: (0, 0)),
            pl.BlockSpec((RB, 1), lambda i: (i, 0)),
        ],
        out_specs=pl.BlockSpec((RB, H), lambda i: (i, 0)),
        out_shape=jax.ShapeDtypeStruct((N, H), jnp.float32),
    )(h, W, a, c, dinv)


def _post_body(acc_ref, hs_ref, dinv_ref, b_ref, h_ref, S_ref, Q_ref):
    u = dinv_ref[...] * (acc_ref[0] + acc_ref[1] - hs_ref[...]) + b_ref[...]
    h = _selu(u)
    h_ref[...] = h

    @pl.when(pl.program_id(0) == 0)
    def _():
        S_ref[...] = jnp.zeros_like(S_ref)
        Q_ref[...] = jnp.zeros_like(Q_ref)

    S_ref[...] += jnp.sum(h, axis=0, keepdims=True)
    Q_ref[...] += jnp.sum(h * h, axis=0, keepdims=True)


def _post(acc, hs, dinv, b):
    return pl.pallas_call(
        _post_body,
        grid=(N // RB,),
        in_specs=[
            pl.BlockSpec((NC, RB, H), lambda i: (0, i, 0)),
            pl.BlockSpec((RB, H), lambda i: (i, 0)),
            pl.BlockSpec((RB, 1), lambda i: (i, 0)),
            pl.BlockSpec((1, H), lambda i: (0, 0)),
        ],
        out_specs=[
            pl.BlockSpec((RB, H), lambda i: (i, 0)),
            pl.BlockSpec((1, H), lambda i: (0, 0)),
            pl.BlockSpec((1, H), lambda i: (0, 0)),
        ],
        out_shape=[
            jax.ShapeDtypeStruct((N, H), jnp.float32),
            jax.ShapeDtypeStruct((1, H), jnp.float32),
            jax.ShapeDtypeStruct((1, H), jnp.float32),
        ],
    )(acc, hs, dinv, b)


def _post3_body(acc_ref, hs_ref, dinv_ref, b_ref, batch_ref, hg_ref):
    u = dinv_ref[...] * (acc_ref[0] + acc_ref[1] - hs_ref[...]) + b_ref[...]
    h = jnp.maximum(u, 0.0)
    onehot = (batch_ref[...] == lax.broadcasted_iota(jnp.int32, (1, G), 1))
    contrib = lax.dot_general(
        onehot.astype(jnp.float32), h,
        dimension_numbers=(((0,), (0,)), ((), ())),
        preferred_element_type=jnp.float32, precision=_PREC)

    @pl.when(pl.program_id(0) == 0)
    def _():
        hg_ref[...] = jnp.zeros_like(hg_ref)

    hg_ref[...] += contrib


def _post3(acc, hs, dinv, b, batch_col):
    return pl.pallas_call(
        _post3_body,
        grid=(N // RB,),
        in_specs=[
            pl.BlockSpec((NC, RB, H), lambda i: (0, i, 0)),
            pl.BlockSpec((RB, H), lambda i: (i, 0)),
            pl.BlockSpec((RB, 1), lambda i: (i, 0)),
            pl.BlockSpec((1, H), lambda i: (0, 0)),
            pl.BlockSpec((RB, 1), lambda i: (i, 0)),
        ],
        out_specs=pl.BlockSpec((G, H), lambda i: (0, 0)),
        out_shape=jax.ShapeDtypeStruct((G, H), jnp.float32),
    )(acc, hs, dinv, b, batch_col)


def _head_body(hg_ref, W1_ref, b1_ref, W2_ref, b2_ref, out_ref):
    hh = _selu(
        jnp.dot(hg_ref[...], W1_ref[...], preferred_element_type=jnp.float32,
                precision=_PREC) + b1_ref[...])
    out_ref[...] = jnp.dot(hh, W2_ref[...], preferred_element_type=jnp.float32,
                           precision=_PREC) + b2_ref[...]


def _head(hg, fc1_W, fc1_b, fc2_W, fc2_b):
    return pl.pallas_call(
        _head_body,
        out_shape=jax.ShapeDtypeStruct((G, DO), jnp.float32),
    )(hg, fc1_W, fc1_b, fc2_W, fc2_b)


def _bn_coeffs(S, Q, gamma, beta):
    mu = S[0] / N
    var = Q[0] / N - mu * mu
    a = gamma * lax.rsqrt(var + 1e-5)
    return a.reshape(1, H), (beta - mu * a).reshape(1, H)


def kernel(x, edge_index, batch, W1, b1, gamma1, beta1, W2, b2, gamma2, beta2,
           W3, b3, fc1_W, fc1_b, fc2_W, fc2_b):
    src = edge_index[0]
    dst = edge_index[1]

    deg2 = _deg_kernel()(dst)
    dinv_row = _dinv(deg2)
    dinv = dinv_row[0, :N].reshape(N, 1)

    one_r = jnp.ones((1, H), jnp.float32)
    zero_r = jnp.zeros((1, H), jnp.float32)

    ps1 = _mm(x, W1, one_r, zero_r, dinv)
    acc1 = _agg_kernel()(ps1, src, dst)
    h1, S1, Q1 = _post(acc1, ps1, dinv, b1.reshape(1, H))
    a1, c1 = _bn_coeffs(S1, Q1, gamma1, beta1)

    ps2 = _mm(h1, W2, a1, c1, dinv)
    acc2 = _agg_kernel()(ps2, src, dst)
    h2, S2, Q2 = _post(acc2, ps2, dinv, b2.reshape(1, H))
    a2, c2 = _bn_coeffs(S2, Q2, gamma2, beta2)

    ps3 = _mm(h2, W3, a2, c2, dinv)
    acc3 = _agg_kernel()(ps3, src, dst)
    hg = _post3(acc3, ps3, dinv, b3.reshape(1, H), batch.reshape(N, 1))

    return _head(hg, fc1_W, fc1_b.reshape(1, DE), fc2_W, fc2_b.reshape(1, DO))


# speedup vs baseline: 25.1297x; 2.2921x over previous
import functools
import jax
import jax.numpy as jnp
from jax import lax
from jax.experimental import pallas as pl
from jax.experimental.pallas import tpu as pltpu
from jax.experimental.pallas import tpu_sc as plsc

N = 10000
E = 320000
F = 128
H = 128
DE = 64
DO = 1
G = 128

NC = 2
NS = 16
NW = NC * NS
EW = E // NW
CH = 80
NCHUNK = EW // CH
RPT = 624
NTAIL = N - NS * RPT
NDEG = 10240
DPT = NDEG // NS

_SELU_ALPHA = 1.6732632423543772
_SELU_SCALE = 1.0507009873554805


def _selu(x):
    return _SELU_SCALE * jnp.where(
        x > 0, x, _SELU_ALPHA * (jnp.exp(jnp.minimum(x, 0.0)) - 1.0))


@functools.cache
def _mesh():
    return plsc.VectorSubcoreMesh(core_axis_name="c", subcore_axis_name="s",
                                  num_cores=NC, num_subcores=NS)


_DEG_LAG = 4


def _deg_body(dst_hbm, out_hbm, idxd_v, ones_v, z_v, acc, sem):
    c = lax.axis_index("c")
    s = lax.axis_index("s")
    wid = s * NC + c

    pltpu.sync_copy(dst_hbm.at[wid], idxd_v)

    @pl.loop(0, DPT, step=16)
    def _(i):
        z_v[pl.ds(i, 16)] = jnp.zeros((16,), jnp.float32)

    pltpu.sync_copy(z_v, acc.at[pl.ds(s * DPT, DPT)])

    @pl.loop(0, CH, step=16)
    def _(i):
        ones_v[pl.ds(i, 16)] = jnp.ones((16,), jnp.float32)

    plsc.subcore_barrier()

    @pl.loop(0, NCHUNK)
    def _(i):
        pltpu.async_copy(ones_v, acc.at[idxd_v.at[i]], sem, add=True)

        @pl.when(i >= _DEG_LAG)
        def _():
            pltpu.make_async_copy(ones_v, acc.at[idxd_v.at[0]], sem).wait()

    for _ in range(_DEG_LAG):
        pltpu.make_async_copy(ones_v, acc.at[idxd_v.at[0]], sem).wait()

    plsc.subcore_barrier()
    pltpu.sync_copy(
        acc.at[pl.ds(s * DPT, DPT)], out_hbm.at[c, pl.ds(s * DPT, DPT)]
    )


@functools.cache
def _deg_kernel():
    return pl.kernel(
        _deg_body,
        out_type=jax.ShapeDtypeStruct((NC, NDEG), jnp.float32),
        mesh=_mesh(),
        scratch_types=[
            pltpu.VMEM((NCHUNK, CH), jnp.int32),
            pltpu.VMEM((CH,), jnp.float32),
            pltpu.VMEM((DPT,), jnp.float32),
            pltpu.VMEM_SHARED((NDEG,), jnp.float32),
            pltpu.SemaphoreType.DMA,
        ],
    )


_NBUF = 2


def _agg_body(hs_hbm, src_hbm, dst_hbm, out_hbm, idxs_v, d0, d1,
              b0, b1, acc, sg0, sg1, sd0, sd1, ss0, ss1):
    c = lax.axis_index("c")
    s = lax.axis_index("s")
    wid = s * NC + c
    bufs = (b0, b1)
    dbufs = (d0, d1)
    sem_g = (sg0, sg1)
    sem_d = (sd0, sd1)
    sem_s = (ss0, ss1)

    pltpu.sync_copy(
        src_hbm.at[pl.ds(pl.multiple_of(wid * EW, 8), EW)], idxs_v)
    pltpu.sync_copy(
        hs_hbm.at[pl.ds(s * RPT, RPT)], acc.at[pl.ds(s * RPT, RPT)]
    )

    @pl.when(s == NS - 1)
    def _():
        pltpu.sync_copy(
            hs_hbm.at[pl.ds(NS * RPT, NTAIL)], acc.at[pl.ds(NS * RPT, NTAIL)]
        )

    plsc.subcore_barrier()

    def issue(k, j):
        pltpu.async_copy(dst_hbm.at[wid, k], dbufs[j], sem_d[j])
        off = pl.multiple_of(k * CH, 8)
        pltpu.async_copy(
            hs_hbm.at[idxs_v.at[pl.ds(off, CH)]], bufs[j], sem_g[j])

    for j in range(_NBUF):
        issue(j, j)

    @pl.loop(0, NCHUNK - 1, step=_NBUF)
    def _(i):
        for j in range(_NBUF):
            pltpu.make_async_copy(
                dst_hbm.at[wid, 0], dbufs[j], sem_d[j]).wait()
            pltpu.make_async_copy(
                hs_hbm.at[idxs_v.at[pl.ds(0, CH)]], bufs[j], sem_g[j]).wait()
            pltpu.async_copy(
                bufs[j], acc.at[dbufs[j]], sem_s[j], add=True).wait()

            @pl.when(i + j + _NBUF < NCHUNK)
            def _():
                issue(i + j + _NBUF, j)

    pltpu.make_async_copy(dst_hbm.at[wid, 0], dbufs[0], sem_d[0]).wait()
    pltpu.make_async_copy(
        hs_hbm.at[idxs_v.at[pl.ds(0, CH)]], bufs[0], sem_g[0]).wait()
    pltpu.async_copy(bufs[0], acc.at[dbufs[0]], sem_s[0], add=True).wait()

    plsc.subcore_barrier()
    pltpu.sync_copy(
        acc.at[pl.ds(s * RPT, RPT)], out_hbm.at[c, pl.ds(s * RPT, RPT)]
    )

    @pl.when(s == NS - 1)
    def _():
        pltpu.sync_copy(
            acc.at[pl.ds(NS * RPT, NTAIL)],
            out_hbm.at[c, pl.ds(NS * RPT, NTAIL)],
        )


@functools.cache
def _agg_kernel():
    return pl.kernel(
        _agg_body,
        out_type=jax.ShapeDtypeStruct((NC, N, H), jnp.float32),
        mesh=_mesh(),
        scratch_types=[
            pltpu.VMEM((EW,), jnp.int32),
            pltpu.VMEM((CH,), jnp.int32),
            pltpu.VMEM((CH,), jnp.int32),
            pltpu.VMEM((CH, H), jnp.float32),
            pltpu.VMEM((CH, H), jnp.float32),
            pltpu.VMEM_SHARED((N, H), jnp.float32),
            pltpu.SemaphoreType.DMA,
            pltpu.SemaphoreType.DMA,
            pltpu.SemaphoreType.DMA,
            pltpu.SemaphoreType.DMA,
            pltpu.SemaphoreType.DMA,
            pltpu.SemaphoreType.DMA,
        ],
    )


RB = 1000
_PREC = lax.Precision.HIGHEST


def _dinv_body(dg_ref, out_ref):
    out_ref[...] = lax.rsqrt(dg_ref[0:1, :] + dg_ref[1:2, :] + 1.0)


def _dinv(deg2):
    return pl.pallas_call(
        _dinv_body,
        out_shape=jax.ShapeDtypeStruct((1, NDEG), jnp.float32),
    )(deg2)


def _mm_body(h_ref, W_ref, a_ref, c_ref, dinv_ref, out_ref):
    hb = h_ref[...] * a_ref[...] + c_ref[...]
    p = jnp.dot(hb, W_ref[...], preferred_element_type=jnp.float32,
                precision=_PREC)
    out_ref[...] = p * dinv_ref[...]


def _mm(h, W, a, c, dinv):
    return pl.pallas_call(
        _mm_body,
        grid=(N // RB,),
        in_specs=[
            pl.BlockSpec((RB, H), lambda i: (i, 0)),
            pl.BlockSpec((H, H), lambda i: (0, 0)),
            pl.BlockSpec((1, H), lambda i: (0, 0)),
            pl.BlockSpec((1, H), lambda i: (0, 0)),
            pl.BlockSpec((RB, 1), lambda i: (i, 0)),
        ],
        out_specs=pl.BlockSpec((RB, H), lambda i: (i, 0)),
        out_shape=jax.ShapeDtypeStruct((N, H), jnp.float32),
    )(h, W, a, c, dinv)


def _post_body(acc_ref, hs_ref, dinv_ref, b_ref, h_ref, S_ref, Q_ref):
    u = dinv_ref[...] * (acc_ref[0] + acc_ref[1] - hs_ref[...]) + b_ref[...]
    h = _selu(u)
    h_ref[...] = h

    @pl.when(pl.program_id(0) == 0)
    def _():
        S_ref[...] = jnp.zeros_like(S_ref)
        Q_ref[...] = jnp.zeros_like(Q_ref)

    S_ref[...] += jnp.sum(h, axis=0, keepdims=True)
    Q_ref[...] += jnp.sum(h * h, axis=0, keepdims=True)


def _post(acc, hs, dinv, b):
    return pl.pallas_call(
        _post_body,
        grid=(N // RB,),
        in_specs=[
            pl.BlockSpec((NC, RB, H), lambda i: (0, i, 0)),
            pl.BlockSpec((RB, H), lambda i: (i, 0)),
            pl.BlockSpec((RB, 1), lambda i: (i, 0)),
            pl.BlockSpec((1, H), lambda i: (0, 0)),
        ],
        out_specs=[
            pl.BlockSpec((RB, H), lambda i: (i, 0)),
            pl.BlockSpec((1, H), lambda i: (0, 0)),
            pl.BlockSpec((1, H), lambda i: (0, 0)),
        ],
        out_shape=[
            jax.ShapeDtypeStruct((N, H), jnp.float32),
            jax.ShapeDtypeStruct((1, H), jnp.float32),
            jax.ShapeDtypeStruct((1, H), jnp.float32),
        ],
    )(acc, hs, dinv, b)


def _post3_body(acc_ref, hs_ref, dinv_ref, b_ref, batch_ref, hg_ref):
    u = dinv_ref[...] * (acc_ref[0] + acc_ref[1] - hs_ref[...]) + b_ref[...]
    h = jnp.maximum(u, 0.0)
    onehot = (batch_ref[...] == lax.broadcasted_iota(jnp.int32, (1, G), 1))
    contrib = lax.dot_general(
        onehot.astype(jnp.float32), h,
        dimension_numbers=(((0,), (0,)), ((), ())),
        preferred_element_type=jnp.float32, precision=_PREC)

    @pl.when(pl.program_id(0) == 0)
    def _():
        hg_ref[...] = jnp.zeros_like(hg_ref)

    hg_ref[...] += contrib


def _post3(acc, hs, dinv, b, batch_col):
    return pl.pallas_call(
        _post3_body,
        grid=(N // RB,),
        in_specs=[
            pl.BlockSpec((NC, RB, H), lambda i: (0, i, 0)),
            pl.BlockSpec((RB, H), lambda i: (i, 0)),
            pl.BlockSpec((RB, 1), lambda i: (i, 0)),
            pl.BlockSpec((1, H), lambda i: (0, 0)),
            pl.BlockSpec((RB, 1), lambda i: (i, 0)),
        ],
        out_specs=pl.BlockSpec((G, H), lambda i: (0, 0)),
        out_shape=jax.ShapeDtypeStruct((G, H), jnp.float32),
    )(acc, hs, dinv, b, batch_col)


def _head_body(hg_ref, W1_ref, b1_ref, W2_ref, b2_ref, out_ref):
    hh = _selu(
        jnp.dot(hg_ref[...], W1_ref[...], preferred_element_type=jnp.float32,
                precision=_PREC) + b1_ref[...])
    out_ref[...] = jnp.dot(hh, W2_ref[...], preferred_element_type=jnp.float32,
                           precision=_PREC) + b2_ref[...]


def _head(hg, fc1_W, fc1_b, fc2_W, fc2_b):
    return pl.pallas_call(
        _head_body,
        out_shape=jax.ShapeDtypeStruct((G, DO), jnp.float32),
    )(hg, fc1_W, fc1_b, fc2_W, fc2_b)


def _bn_coeffs(S, Q, gamma, beta):
    mu = S[0] / N
    var = Q[0] / N - mu * mu
    a = gamma * lax.rsqrt(var + 1e-5)
    return a.reshape(1, H), (beta - mu * a).reshape(1, H)


def kernel(x, edge_index, batch, W1, b1, gamma1, beta1, W2, b2, gamma2, beta2,
           W3, b3, fc1_W, fc1_b, fc2_W, fc2_b):
    src = edge_index[0]
    dst = edge_index[1].reshape(NW, NCHUNK, CH)

    deg2 = _deg_kernel()(dst)
    dinv_row = _dinv(deg2)
    dinv = dinv_row[0, :N].reshape(N, 1)

    one_r = jnp.ones((1, H), jnp.float32)
    zero_r = jnp.zeros((1, H), jnp.float32)

    ps1 = _mm(x, W1, one_r, zero_r, dinv)
    acc1 = _agg_kernel()(ps1, src, dst)
    h1, S1, Q1 = _post(acc1, ps1, dinv, b1.reshape(1, H))
    a1, c1 = _bn_coeffs(S1, Q1, gamma1, beta1)

    ps2 = _mm(h1, W2, a1, c1, dinv)
    acc2 = _agg_kernel()(ps2, src, dst)
    h2, S2, Q2 = _post(acc2, ps2, dinv, b2.reshape(1, H))
    a2, c2 = _bn_coeffs(S2, Q2, gamma2, beta2)

    ps3 = _mm(h2, W3, a2, c2, dinv)
    acc3 = _agg_kernel()(ps3, src, dst)
    hg = _post3(acc3, ps3, dinv, b3.reshape(1, H), batch.reshape(N, 1))

    return _head(hg, fc1_W, fc1_b.reshape(1, DE), fc2_W, fc2_b.reshape(1, DO))


# speedup vs baseline: 29.0407x; 1.1556x over previous
import functools
import jax
import jax.numpy as jnp
from jax import lax
from jax.experimental import pallas as pl
from jax.experimental.pallas import tpu as pltpu
from jax.experimental.pallas import tpu_sc as plsc

N = 10000
E = 320000
F = 128
H = 128
DE = 64
DO = 1
G = 128

NC = 2
NS = 16
NW = NC * NS
EW = E // NW
CH = 80
NCHUNK = EW // CH
RPT = 624
NTAIL = N - NS * RPT
NDEG = 10240
DPT = NDEG // NS

_SELU_ALPHA = 1.6732632423543772
_SELU_SCALE = 1.0507009873554805


def _selu(x):
    return _SELU_SCALE * jnp.where(
        x > 0, x, _SELU_ALPHA * (jnp.exp(jnp.minimum(x, 0.0)) - 1.0))


@functools.cache
def _mesh():
    return plsc.VectorSubcoreMesh(core_axis_name="c", subcore_axis_name="s",
                                  num_cores=NC, num_subcores=NS)


_DEG_LAG = 4


def _deg_body(dst_hbm, out_hbm, idxd_v, ones_v, z_v, acc, sem):
    c = lax.axis_index("c")
    s = lax.axis_index("s")
    wid = s * NC + c

    pltpu.sync_copy(dst_hbm.at[wid], idxd_v)

    @pl.loop(0, DPT, step=16)
    def _(i):
        z_v[pl.ds(i, 16)] = jnp.zeros((16,), jnp.float32)

    pltpu.sync_copy(z_v, acc.at[pl.ds(s * DPT, DPT)])

    @pl.loop(0, CH, step=16)
    def _(i):
        ones_v[pl.ds(i, 16)] = jnp.ones((16,), jnp.float32)

    plsc.subcore_barrier()

    @pl.loop(0, NCHUNK)
    def _(i):
        pltpu.async_copy(ones_v, acc.at[idxd_v.at[i]], sem, add=True)

        @pl.when(i >= _DEG_LAG)
        def _():
            pltpu.make_async_copy(ones_v, acc.at[idxd_v.at[0]], sem).wait()

    for _ in range(_DEG_LAG):
        pltpu.make_async_copy(ones_v, acc.at[idxd_v.at[0]], sem).wait()

    plsc.subcore_barrier()
    pltpu.sync_copy(
        acc.at[pl.ds(s * DPT, DPT)], out_hbm.at[c, pl.ds(s * DPT, DPT)]
    )


@functools.cache
def _deg_kernel():
    return pl.kernel(
        _deg_body,
        out_type=jax.ShapeDtypeStruct((NC, NDEG), jnp.float32),
        mesh=_mesh(),
        scratch_types=[
            pltpu.VMEM((NCHUNK, CH), jnp.int32),
            pltpu.VMEM((CH,), jnp.float32),
            pltpu.VMEM((DPT,), jnp.float32),
            pltpu.VMEM_SHARED((NDEG,), jnp.float32),
            pltpu.SemaphoreType.DMA,
        ],
    )


_NBUF = 3


def _agg_body(hs_hbm, src_hbm, dst_hbm, out_hbm, idxs_v, d0, d1, d2,
              b0, b1, b2, acc, sg0, sg1, sg2, sd0, sd1, sd2, ss0, ss1, ss2):
    c = lax.axis_index("c")
    s = lax.axis_index("s")
    wid = s * NC + c
    bufs = (b0, b1, b2)
    dbufs = (d0, d1, d2)
    sem_g = (sg0, sg1, sg2)
    sem_d = (sd0, sd1, sd2)
    sem_s = (ss0, ss1, ss2)

    pltpu.sync_copy(
        src_hbm.at[pl.ds(pl.multiple_of(wid * EW, 8), EW)], idxs_v)
    pltpu.sync_copy(
        hs_hbm.at[pl.ds(s * RPT, RPT)], acc.at[pl.ds(s * RPT, RPT)]
    )

    @pl.when(s == NS - 1)
    def _():
        pltpu.sync_copy(
            hs_hbm.at[pl.ds(NS * RPT, NTAIL)], acc.at[pl.ds(NS * RPT, NTAIL)]
        )

    plsc.subcore_barrier()

    def issue(k, j):
        pltpu.async_copy(dst_hbm.at[wid, k], dbufs[j], sem_d[j])
        off = pl.multiple_of(k * CH, 8)
        pltpu.async_copy(
            hs_hbm.at[idxs_v.at[pl.ds(off, CH)]], bufs[j], sem_g[j])

    def drain_and_scatter(j):
        pltpu.make_async_copy(dst_hbm.at[wid, 0], dbufs[j], sem_d[j]).wait()
        pltpu.make_async_copy(
            hs_hbm.at[idxs_v.at[pl.ds(0, CH)]], bufs[j], sem_g[j]).wait()
        pltpu.async_copy(bufs[j], acc.at[dbufs[j]], sem_s[j], add=True).wait()

    for j in range(_NBUF):
        issue(j, j)

    _T = ((NCHUNK - 1) // _NBUF) * _NBUF

    @pl.loop(0, _T, step=_NBUF)
    def _(i):
        for j in range(_NBUF):
            drain_and_scatter(j)

            @pl.when(i + j + _NBUF < NCHUNK)
            def _():
                issue(i + j + _NBUF, j)

    for k in range(_T, NCHUNK):
        drain_and_scatter(k % _NBUF)

    plsc.subcore_barrier()
    pltpu.sync_copy(
        acc.at[pl.ds(s * RPT, RPT)], out_hbm.at[c, pl.ds(s * RPT, RPT)]
    )

    @pl.when(s == NS - 1)
    def _():
        pltpu.sync_copy(
            acc.at[pl.ds(NS * RPT, NTAIL)],
            out_hbm.at[c, pl.ds(NS * RPT, NTAIL)],
        )


@functools.cache
def _agg_kernel():
    return pl.kernel(
        _agg_body,
        out_type=jax.ShapeDtypeStruct((NC, N, H), jnp.float32),
        mesh=_mesh(),
        scratch_types=(
            [pltpu.VMEM((EW,), jnp.int32)]
            + [pltpu.VMEM((CH,), jnp.int32)] * _NBUF
            + [pltpu.VMEM((CH, H), jnp.float32)] * _NBUF
            + [pltpu.VMEM_SHARED((N, H), jnp.float32)]
            + [pltpu.SemaphoreType.DMA] * (3 * _NBUF)
        ),
    )


RB = 1000
_PREC = lax.Precision.HIGHEST


def _dinv_body(dg_ref, out_ref):
    out_ref[...] = lax.rsqrt(dg_ref[0:1, :] + dg_ref[1:2, :] + 1.0)


def _dinv(deg2):
    return pl.pallas_call(
        _dinv_body,
        out_shape=jax.ShapeDtypeStruct((1, NDEG), jnp.float32),
    )(deg2)


def _mm_body(h_ref, W_ref, a_ref, c_ref, dinv_ref, out_ref):
    hb = h_ref[...] * a_ref[...] + c_ref[...]
    p = jnp.dot(hb, W_ref[...], preferred_element_type=jnp.float32,
                precision=_PREC)
    out_ref[...] = p * dinv_ref[...]


def _mm(h, W, a, c, dinv):
    return pl.pallas_call(
        _mm_body,
        grid=(N // RB,),
        in_specs=[
            pl.BlockSpec((RB, H), lambda i: (i, 0)),
            pl.BlockSpec((H, H), lambda i: (0, 0)),
            pl.BlockSpec((1, H), lambda i: (0, 0)),
            pl.BlockSpec((1, H), lambda i: (0, 0)),
            pl.BlockSpec((RB, 1), lambda i: (i, 0)),
        ],
        out_specs=pl.BlockSpec((RB, H), lambda i: (i, 0)),
        out_shape=jax.ShapeDtypeStruct((N, H), jnp.float32),
    )(h, W, a, c, dinv)


def _post_body(acc_ref, hs_ref, dinv_ref, b_ref, h_ref, S_ref, Q_ref):
    u = dinv_ref[...] * (acc_ref[0] + acc_ref[1] - hs_ref[...]) + b_ref[...]
    h = _selu(u)
    h_ref[...] = h

    @pl.when(pl.program_id(0) == 0)
    def _():
        S_ref[...] = jnp.zeros_like(S_ref)
        Q_ref[...] = jnp.zeros_like(Q_ref)

    S_ref[...] += jnp.sum(h, axis=0, keepdims=True)
    Q_ref[...] += jnp.sum(h * h, axis=0, keepdims=True)


def _post(acc, hs, dinv, b):
    return pl.pallas_call(
        _post_body,
        grid=(N // RB,),
        in_specs=[
            pl.BlockSpec((NC, RB, H), lambda i: (0, i, 0)),
            pl.BlockSpec((RB, H), lambda i: (i, 0)),
            pl.BlockSpec((RB, 1), lambda i: (i, 0)),
            pl.BlockSpec((1, H), lambda i: (0, 0)),
        ],
        out_specs=[
            pl.BlockSpec((RB, H), lambda i: (i, 0)),
            pl.BlockSpec((1, H), lambda i: (0, 0)),
            pl.BlockSpec((1, H), lambda i: (0, 0)),
        ],
        out_shape=[
            jax.ShapeDtypeStruct((N, H), jnp.float32),
            jax.ShapeDtypeStruct((1, H), jnp.float32),
            jax.ShapeDtypeStruct((1, H), jnp.float32),
        ],
    )(acc, hs, dinv, b)


def _post3_body(acc_ref, hs_ref, dinv_ref, b_ref, batch_ref, hg_ref):
    u = dinv_ref[...] * (acc_ref[0] + acc_ref[1] - hs_ref[...]) + b_ref[...]
    h = jnp.maximum(u, 0.0)
    onehot = (batch_ref[...] == lax.broadcasted_iota(jnp.int32, (1, G), 1))
    contrib = lax.dot_general(
        onehot.astype(jnp.float32), h,
        dimension_numbers=(((0,), (0,)), ((), ())),
        preferred_element_type=jnp.float32, precision=_PREC)

    @pl.when(pl.program_id(0) == 0)
    def _():
        hg_ref[...] = jnp.zeros_like(hg_ref)

    hg_ref[...] += contrib


def _post3(acc, hs, dinv, b, batch_col):
    return pl.pallas_call(
        _post3_body,
        grid=(N // RB,),
        in_specs=[
            pl.BlockSpec((NC, RB, H), lambda i: (0, i, 0)),
            pl.BlockSpec((RB, H), lambda i: (i, 0)),
            pl.BlockSpec((RB, 1), lambda i: (i, 0)),
            pl.BlockSpec((1, H), lambda i: (0, 0)),
            pl.BlockSpec((RB, 1), lambda i: (i, 0)),
        ],
        out_specs=pl.BlockSpec((G, H), lambda i: (0, 0)),
        out_shape=jax.ShapeDtypeStruct((G, H), jnp.float32),
    )(acc, hs, dinv, b, batch_col)


def _head_body(hg_ref, W1_ref, b1_ref, W2_ref, b2_ref, out_ref):
    hh = _selu(
        jnp.dot(hg_ref[...], W1_ref[...], preferred_element_type=jnp.float32,
                precision=_PREC) + b1_ref[...])
    out_ref[...] = jnp.dot(hh, W2_ref[...], preferred_element_type=jnp.float32,
                           precision=_PREC) + b2_ref[...]


def _head(hg, fc1_W, fc1_b, fc2_W, fc2_b):
    return pl.pallas_call(
        _head_body,
        out_shape=jax.ShapeDtypeStruct((G, DO), jnp.float32),
    )(hg, fc1_W, fc1_b, fc2_W, fc2_b)


def _bn_coeffs(S, Q, gamma, beta):
    mu = S[0] / N
    var = Q[0] / N - mu * mu
    a = gamma * lax.rsqrt(var + 1e-5)
    return a.reshape(1, H), (beta - mu * a).reshape(1, H)


def kernel(x, edge_index, batch, W1, b1, gamma1, beta1, W2, b2, gamma2, beta2,
           W3, b3, fc1_W, fc1_b, fc2_W, fc2_b):
    src = edge_index[0]
    dst = edge_index[1].reshape(NW, NCHUNK, CH)

    deg2 = _deg_kernel()(dst)
    dinv_row = _dinv(deg2)
    dinv = dinv_row[0, :N].reshape(N, 1)

    one_r = jnp.ones((1, H), jnp.float32)
    zero_r = jnp.zeros((1, H), jnp.float32)

    ps1 = _mm(x, W1, one_r, zero_r, dinv)
    acc1 = _agg_kernel()(ps1, src, dst)
    h1, S1, Q1 = _post(acc1, ps1, dinv, b1.reshape(1, H))
    a1, c1 = _bn_coeffs(S1, Q1, gamma1, beta1)

    ps2 = _mm(h1, W2, a1, c1, dinv)
    acc2 = _agg_kernel()(ps2, src, dst)
    h2, S2, Q2 = _post(acc2, ps2, dinv, b2.reshape(1, H))
    a2, c2 = _bn_coeffs(S2, Q2, gamma2, beta2)

    ps3 = _mm(h2, W3, a2, c2, dinv)
    acc3 = _agg_kernel()(ps3, src, dst)
    hg = _post3(acc3, ps3, dinv, b3.reshape(1, H), batch.reshape(N, 1))

    return _head(hg, fc1_W, fc1_b.reshape(1, DE), fc2_W, fc2_b.reshape(1, DO))


# speedup vs baseline: 29.4605x; 1.0145x over previous
import functools
import jax
import jax.numpy as jnp
from jax import lax
from jax.experimental import pallas as pl
from jax.experimental.pallas import tpu as pltpu
from jax.experimental.pallas import tpu_sc as plsc

N = 10000
E = 320000
F = 128
H = 128
DE = 64
DO = 1
G = 128

NC = 2
NS = 16
NW = NC * NS
EW = E // NW
CH = 40
NCHUNK = EW // CH
RPT = 624
NTAIL = N - NS * RPT
NDEG = 10240
DPT = NDEG // NS

_SELU_ALPHA = 1.6732632423543772
_SELU_SCALE = 1.0507009873554805


def _selu(x):
    return _SELU_SCALE * jnp.where(
        x > 0, x, _SELU_ALPHA * (jnp.exp(jnp.minimum(x, 0.0)) - 1.0))


@functools.cache
def _mesh():
    return plsc.VectorSubcoreMesh(core_axis_name="c", subcore_axis_name="s",
                                  num_cores=NC, num_subcores=NS)


_DEG_LAG = 4


def _deg_body(dst_hbm, out_hbm, idxd_v, ones_v, z_v, acc, sem):
    c = lax.axis_index("c")
    s = lax.axis_index("s")
    wid = s * NC + c

    pltpu.sync_copy(dst_hbm.at[wid], idxd_v)

    @pl.loop(0, DPT, step=16)
    def _(i):
        z_v[pl.ds(i, 16)] = jnp.zeros((16,), jnp.float32)

    pltpu.sync_copy(z_v, acc.at[pl.ds(s * DPT, DPT)])

    @pl.loop(0, CH, step=16)
    def _(i):
        ones_v[pl.ds(i, 16)] = jnp.ones((16,), jnp.float32)

    plsc.subcore_barrier()

    @pl.loop(0, NCHUNK)
    def _(i):
        pltpu.async_copy(ones_v, acc.at[idxd_v.at[i]], sem, add=True)

        @pl.when(i >= _DEG_LAG)
        def _():
            pltpu.make_async_copy(ones_v, acc.at[idxd_v.at[0]], sem).wait()

    for _ in range(_DEG_LAG):
        pltpu.make_async_copy(ones_v, acc.at[idxd_v.at[0]], sem).wait()

    plsc.subcore_barrier()
    pltpu.sync_copy(
        acc.at[pl.ds(s * DPT, DPT)], out_hbm.at[c, pl.ds(s * DPT, DPT)]
    )


@functools.cache
def _deg_kernel():
    return pl.kernel(
        _deg_body,
        out_type=jax.ShapeDtypeStruct((NC, NDEG), jnp.float32),
        mesh=_mesh(),
        scratch_types=[
            pltpu.VMEM((NCHUNK, CH), jnp.int32),
            pltpu.VMEM((CH,), jnp.float32),
            pltpu.VMEM((DPT,), jnp.float32),
            pltpu.VMEM_SHARED((NDEG,), jnp.float32),
            pltpu.SemaphoreType.DMA,
        ],
    )


_NBUF = 6


def _agg_body(hs_hbm, src_hbm, dst_hbm, out_hbm, idxs_v, d0, d1, d2, d3, d4,
              d5, b0, b1, b2, b3, b4, b5, acc, sg0, sg1, sg2, sg3, sg4, sg5,
              sd0, sd1, sd2, sd3, sd4, sd5, ss0, ss1, ss2, ss3, ss4, ss5):
    c = lax.axis_index("c")
    s = lax.axis_index("s")
    wid = s * NC + c
    bufs = (b0, b1, b2, b3, b4, b5)
    dbufs = (d0, d1, d2, d3, d4, d5)
    sem_g = (sg0, sg1, sg2, sg3, sg4, sg5)
    sem_d = (sd0, sd1, sd2, sd3, sd4, sd5)
    sem_s = (ss0, ss1, ss2, ss3, ss4, ss5)

    pltpu.sync_copy(
        src_hbm.at[pl.ds(pl.multiple_of(wid * EW, 8), EW)], idxs_v)
    pltpu.sync_copy(
        hs_hbm.at[pl.ds(s * RPT, RPT)], acc.at[pl.ds(s * RPT, RPT)]
    )

    @pl.when(s == NS - 1)
    def _():
        pltpu.sync_copy(
            hs_hbm.at[pl.ds(NS * RPT, NTAIL)], acc.at[pl.ds(NS * RPT, NTAIL)]
        )

    plsc.subcore_barrier()

    def issue(k, j):
        pltpu.async_copy(dst_hbm.at[wid, k], dbufs[j], sem_d[j])
        off = pl.multiple_of(k * CH, 8)
        pltpu.async_copy(
            hs_hbm.at[idxs_v.at[pl.ds(off, CH)]], bufs[j], sem_g[j])

    def drain_and_scatter(j):
        pltpu.make_async_copy(dst_hbm.at[wid, 0], dbufs[j], sem_d[j]).wait()
        pltpu.make_async_copy(
            hs_hbm.at[idxs_v.at[pl.ds(0, CH)]], bufs[j], sem_g[j]).wait()
        pltpu.async_copy(bufs[j], acc.at[dbufs[j]], sem_s[j], add=True).wait()

    for j in range(_NBUF):
        issue(j, j)

    _T = ((NCHUNK - 1) // _NBUF) * _NBUF

    @pl.loop(0, _T, step=_NBUF)
    def _(i):
        for j in range(_NBUF):
            drain_and_scatter(j)

            @pl.when(i + j + _NBUF < NCHUNK)
            def _():
                issue(i + j + _NBUF, j)

    for k in range(_T, NCHUNK):
        drain_and_scatter(k % _NBUF)

    plsc.subcore_barrier()
    pltpu.sync_copy(
        acc.at[pl.ds(s * RPT, RPT)], out_hbm.at[c, pl.ds(s * RPT, RPT)]
    )

    @pl.when(s == NS - 1)
    def _():
        pltpu.sync_copy(
            acc.at[pl.ds(NS * RPT, NTAIL)],
            out_hbm.at[c, pl.ds(NS * RPT, NTAIL)],
        )


@functools.cache
def _agg_kernel():
    return pl.kernel(
        _agg_body,
        out_type=jax.ShapeDtypeStruct((NC, N, H), jnp.float32),
        mesh=_mesh(),
        scratch_types=(
            [pltpu.VMEM((EW,), jnp.int32)]
            + [pltpu.VMEM((CH,), jnp.int32)] * _NBUF
            + [pltpu.VMEM((CH, H), jnp.float32)] * _NBUF
            + [pltpu.VMEM_SHARED((N, H), jnp.float32)]
            + [pltpu.SemaphoreType.DMA] * (3 * _NBUF)
        ),
    )


RB = 1000
_PREC = lax.Precision.HIGHEST


def _dinv_body(dg_ref, out_ref):
    out_ref[...] = lax.rsqrt(dg_ref[0:1, :] + dg_ref[1:2, :] + 1.0)


def _dinv(deg2):
    return pl.pallas_call(
        _dinv_body,
        out_shape=jax.ShapeDtypeStruct((1, NDEG), jnp.float32),
    )(deg2)


def _mm_body(h_ref, W_ref, a_ref, c_ref, dinv_ref, out_ref):
    hb = h_ref[...] * a_ref[...] + c_ref[...]
    p = jnp.dot(hb, W_ref[...], preferred_element_type=jnp.float32,
                precision=_PREC)
    out_ref[...] = p * dinv_ref[...]


def _mm(h, W, a, c, dinv):
    return pl.pallas_call(
        _mm_body,
        grid=(N // RB,),
        in_specs=[
            pl.BlockSpec((RB, H), lambda i: (i, 0)),
            pl.BlockSpec((H, H), lambda i: (0, 0)),
            pl.BlockSpec((1, H), lambda i: (0, 0)),
            pl.BlockSpec((1, H), lambda i: (0, 0)),
            pl.BlockSpec((RB, 1), lambda i: (i, 0)),
        ],
        out_specs=pl.BlockSpec((RB, H), lambda i: (i, 0)),
        out_shape=jax.ShapeDtypeStruct((N, H), jnp.float32),
    )(h, W, a, c, dinv)


def _post_body(acc_ref, hs_ref, dinv_ref, b_ref, h_ref, S_ref, Q_ref):
    u = dinv_ref[...] * (acc_ref[0] + acc_ref[1] - hs_ref[...]) + b_ref[...]
    h = _selu(u)
    h_ref[...] = h

    @pl.when(pl.program_id(0) == 0)
    def _():
        S_ref[...] = jnp.zeros_like(S_ref)
        Q_ref[...] = jnp.zeros_like(Q_ref)

    S_ref[...] += jnp.sum(h, axis=0, keepdims=True)
    Q_ref[...] += jnp.sum(h * h, axis=0, keepdims=True)


def _post(acc, hs, dinv, b):
    return pl.pallas_call(
        _post_body,
        grid=(N // RB,),
        in_specs=[
            pl.BlockSpec((NC, RB, H), lambda i: (0, i, 0)),
            pl.BlockSpec((RB, H), lambda i: (i, 0)),
            pl.BlockSpec((RB, 1), lambda i: (i, 0)),
            pl.BlockSpec((1, H), lambda i: (0, 0)),
        ],
        out_specs=[
            pl.BlockSpec((RB, H), lambda i: (i, 0)),
            pl.BlockSpec((1, H), lambda i: (0, 0)),
            pl.BlockSpec((1, H), lambda i: (0, 0)),
        ],
        out_shape=[
            jax.ShapeDtypeStruct((N, H), jnp.float32),
            jax.ShapeDtypeStruct((1, H), jnp.float32),
            jax.ShapeDtypeStruct((1, H), jnp.float32),
        ],
    )(acc, hs, dinv, b)


def _post3_body(acc_ref, hs_ref, dinv_ref, b_ref, batch_ref, hg_ref):
    u = dinv_ref[...] * (acc_ref[0] + acc_ref[1] - hs_ref[...]) + b_ref[...]
    h = jnp.maximum(u, 0.0)
    onehot = (batch_ref[...] == lax.broadcasted_iota(jnp.int32, (1, G), 1))
    contrib = lax.dot_general(
        onehot.astype(jnp.float32), h,
        dimension_numbers=(((0,), (0,)), ((), ())),
        preferred_element_type=jnp.float32, precision=_PREC)

    @pl.when(pl.program_id(0) == 0)
    def _():
        hg_ref[...] = jnp.zeros_like(hg_ref)

    hg_ref[...] += contrib


def _post3(acc, hs, dinv, b, batch_col):
    return pl.pallas_call(
        _post3_body,
        grid=(N // RB,),
        in_specs=[
            pl.BlockSpec((NC, RB, H), lambda i: (0, i, 0)),
            pl.BlockSpec((RB, H), lambda i: (i, 0)),
            pl.BlockSpec((RB, 1), lambda i: (i, 0)),
            pl.BlockSpec((1, H), lambda i: (0, 0)),
            pl.BlockSpec((RB, 1), lambda i: (i, 0)),
        ],
        out_specs=pl.BlockSpec((G, H), lambda i: (0, 0)),
        out_shape=jax.ShapeDtypeStruct((G, H), jnp.float32),
    )(acc, hs, dinv, b, batch_col)


def _head_body(hg_ref, W1_ref, b1_ref, W2_ref, b2_ref, out_ref):
    hh = _selu(
        jnp.dot(hg_ref[...], W1_ref[...], preferred_element_type=jnp.float32,
                precision=_PREC) + b1_ref[...])
    out_ref[...] = jnp.dot(hh, W2_ref[...], preferred_element_type=jnp.float32,
                           precision=_PREC) + b2_ref[...]


def _head(hg, fc1_W, fc1_b, fc2_W, fc2_b):
    return pl.pallas_call(
        _head_body,
        out_shape=jax.ShapeDtypeStruct((G, DO), jnp.float32),
    )(hg, fc1_W, fc1_b, fc2_W, fc2_b)


def _bn_coeffs(S, Q, gamma, beta):
    mu = S[0] / N
    var = Q[0] / N - mu * mu
    a = gamma * lax.rsqrt(var + 1e-5)
    return a.reshape(1, H), (beta - mu * a).reshape(1, H)


def kernel(x, edge_index, batch, W1, b1, gamma1, beta1, W2, b2, gamma2, beta2,
           W3, b3, fc1_W, fc1_b, fc2_W, fc2_b):
    src = edge_index[0]
    dst = edge_index[1].reshape(NW, NCHUNK, CH)

    deg2 = _deg_kernel()(dst)
    dinv_row = _dinv(deg2)
    dinv = dinv_row[0, :N].reshape(N, 1)

    one_r = jnp.ones((1, H), jnp.float32)
    zero_r = jnp.zeros((1, H), jnp.float32)

    ps1 = _mm(x, W1, one_r, zero_r, dinv)
    acc1 = _agg_kernel()(ps1, src, dst)
    h1, S1, Q1 = _post(acc1, ps1, dinv, b1.reshape(1, H))
    a1, c1 = _bn_coeffs(S1, Q1, gamma1, beta1)

    ps2 = _mm(h1, W2, a1, c1, dinv)
    acc2 = _agg_kernel()(ps2, src, dst)
    h2, S2, Q2 = _post(acc2, ps2, dinv, b2.reshape(1, H))
    a2, c2 = _bn_coeffs(S2, Q2, gamma2, beta2)

    ps3 = _mm(h2, W3, a2, c2, dinv)
    acc3 = _agg_kernel()(ps3, src, dst)
    hg = _post3(acc3, ps3, dinv, b3.reshape(1, H), batch.reshape(N, 1))

    return _head(hg, fc1_W, fc1_b.reshape(1, DE), fc2_W, fc2_b.reshape(1, DO))
